# baseline (speedup 1.0000x reference)
import functools

import jax
import jax.numpy as jnp
from jax import lax
from jax.experimental import pallas as pl
from jax.experimental.pallas import tpu as pltpu
from jax.experimental.pallas import tpu_sc as plsc

B = 16
L = 4096
D = 8
NC = 2
NS = 16
HALF = L // 2
OUT_W = HALF * D
SECS_PER_DAY = 86400

_mesh = plsc.VectorSubcoreMesh(core_axis_name="c", subcore_axis_name="s")


@functools.partial(
    pl.kernel,
    mesh=_mesh,
    out_type=jax.ShapeDtypeStruct((B, L * D), jnp.float32),
    scratch_types=[
        pltpu.VMEM((L,), jnp.int32),
        pltpu.VMEM((L,), jnp.float32),
        pltpu.VMEM((16,), jnp.int32),
        pltpu.VMEM((16,), jnp.float32),
        pltpu.VMEM((16,), jnp.float32),
        pltpu.VMEM((16,), jnp.float32),
        pltpu.VMEM((OUT_W,), jnp.float32),
    ],
    compiler_params=pltpu.CompilerParams(needs_layout_passes=False),
)
def _sc_embed(ts_hbm, len_hbm, wv_hbm, bv_hbm, out_hbm,
              ts_v, secs_v, len_v, wv_v, bv_v, red_v, out_v):
    wid = lax.axis_index("s") * NC + lax.axis_index("c")
    row = wid // 2
    half = wid % 2

    pltpu.sync_copy(ts_hbm.at[row], ts_v)
    pltpu.sync_copy(len_hbm, len_v)
    pltpu.sync_copy(wv_hbm, wv_v)
    pltpu.sync_copy(bv_hbm, bv_v)

    iota = lax.iota(jnp.int32, 16)
    lane_pair = lax.shift_right_logical(iota, 3)

    def pass1(j, carry):
        mn_v, mx_v = carry
        ts = ts_v[pl.ds(j * 16, 16)]
        secs = lax.rem(ts, SECS_PER_DAY).astype(jnp.float32)
        secs_v[pl.ds(j * 16, 16)] = secs
        return jnp.minimum(mn_v, secs), jnp.maximum(mx_v, secs)

    inf_v = jnp.full((16,), jnp.inf, jnp.float32)
    mn_v, mx_v = lax.fori_loop(0, L // 16, pass1, (inf_v, -inf_v))

    def lane_all_reduce(v, op):
        for step in (8, 4, 2, 1):
            red_v[...] = v
            v = op(v, plsc.load_gather(red_v, [lax.bitwise_xor(iota, step)]))
        return v

    mn = lane_all_reduce(mn_v, jnp.minimum)
    mx = lane_all_reduce(mx_v, jnp.maximum)

    wv = wv_v[...]
    bv = bv_v[...]
    scale = wv / (mx - mn)
    row_len = plsc.load_gather(len_v, [jnp.broadcast_to(row, (16,))])

    base = half * HALF

    def pass2(j, _):
        for t in range(8):
            l_idx = lane_pair + (base + j * 16 + 2 * t)
            s_g = plsc.load_gather(secs_v, [l_idx])
            o = (s_g - mn) * scale + bv
            o = jnp.where(l_idx < row_len, o, 0.0)
            out_v[pl.ds(j * 128 + t * 16, 16)] = o
        return 0

    lax.fori_loop(0, HALF // 16, pass2, 0)
    pltpu.sync_copy(out_v, out_hbm.at[row, pl.ds(half * OUT_W, OUT_W)])


@jax.jit
def kernel(time_seqs, seq_lengths, W, b):
    ts = time_seqs.astype(jnp.int32)
    sl = seq_lengths.astype(jnp.int32)
    wv = jnp.tile(W.reshape(-1), 2)
    bv = jnp.tile(b.reshape(-1), 2)
    out = _sc_embed(ts, sl, wv, bv)
    return out.reshape(B, L, D)


# speedup vs baseline: 1.2759x; 1.2759x over previous
import functools

import jax
import jax.numpy as jnp
from jax import lax
from jax.experimental import pallas as pl
from jax.experimental.pallas import tpu as pltpu
from jax.experimental.pallas import tpu_sc as plsc

B = 16
L = 4096
D = 8
NC = 2
HALF = L // 2
OUT_W = HALF * D
N_CHUNKS = 4
CHUNK_BLKS = (HALF // 16) // N_CHUNKS
CHUNK_W = CHUNK_BLKS * 128

_mesh = plsc.VectorSubcoreMesh(core_axis_name="c", subcore_axis_name="s")


@functools.partial(
    pl.kernel,
    mesh=_mesh,
    out_type=jax.ShapeDtypeStruct((B, L * D), jnp.float32),
    scratch_types=[
        pltpu.VMEM((L,), jnp.int32),
        pltpu.VMEM((L,), jnp.float32),
        pltpu.VMEM((16,), jnp.int32),
        pltpu.VMEM((32,), jnp.float32),
        pltpu.VMEM((16,), jnp.float32),
        pltpu.VMEM((OUT_W,), jnp.float32),
        pltpu.SemaphoreType.DMA,
    ],
    compiler_params=pltpu.CompilerParams(needs_layout_passes=False),
)
def _sc_embed(ts_hbm, len_hbm, wb_hbm, out_hbm,
              ts_v, secs_v, len_v, wb_v, red_v, out_v, sem):
    wid = lax.axis_index("s") * NC + lax.axis_index("c")
    row = wid // 2
    half = wid % 2

    c1 = pltpu.async_copy(ts_hbm.at[row], ts_v, sem)
    c2 = pltpu.async_copy(len_hbm, len_v, sem)
    c3 = pltpu.async_copy(wb_hbm, wb_v, sem)
    c1.wait()
    c2.wait()
    c3.wait()

    iota = lax.iota(jnp.int32, 16)
    lane_pair = lax.shift_right_logical(iota, 3)
    inv_day = jnp.float32(1.0 / 86400.0)

    inf_v = jnp.full((16,), jnp.inf, jnp.float32)

    @plsc.parallel_loop(0, L // 16, unroll=8, carry=(inf_v, -inf_v))
    def _pass1(j, carry):
        mn_v, mx_v = carry
        ts = ts_v[pl.ds(j * 16, 16)]
        xf = ts.astype(jnp.float32)
        q = (xf * inv_day).astype(jnp.int32)
        qm = ((q.astype(jnp.float32) * 675.0) * 128.0).astype(jnp.int32)
        r = ts - qm
        r = jnp.where(r < 0, r + 86400, r)
        r = jnp.where(r >= 86400, r - 86400, r)
        secs = r.astype(jnp.float32)
        secs_v[pl.ds(j * 16, 16)] = secs
        return jnp.minimum(mn_v, secs), jnp.maximum(mx_v, secs)

    mn_v, mx_v = _pass1

    def lane_all_reduce(v, op):
        for step in (8, 4, 2, 1):
            red_v[...] = v
            v = op(v, plsc.load_gather(red_v, [lax.bitwise_xor(iota, step)]))
        return v

    mn = lane_all_reduce(mn_v, jnp.minimum)
    mx = lane_all_reduce(mx_v, jnp.maximum)

    wv = wb_v[pl.ds(0, 16)]
    bv = wb_v[pl.ds(16, 16)]
    scale = wv / (mx - mn)
    bias = bv - mn * scale
    row_len = plsc.load_gather(len_v, [jnp.broadcast_to(row, (16,))])

    base = half * HALF

    copies = []
    for c in range(N_CHUNKS):

        @plsc.parallel_loop(c * CHUNK_BLKS, (c + 1) * CHUNK_BLKS, unroll=2)
        def _pass2(j):
            lj = base + j * 16
            for t in range(8):
                l_idx = lane_pair + (lj + 2 * t)
                s_g = plsc.load_gather(secs_v, [l_idx])
                o = s_g * scale + bias
                o = jnp.where(l_idx < row_len, o, 0.0)
                out_v[pl.ds(j * 128 + t * 16, 16)] = o

        copies.append(pltpu.async_copy(
            out_v.at[pl.ds(c * CHUNK_W, CHUNK_W)],
            out_hbm.at[row, pl.ds(half * OUT_W + c * CHUNK_W, CHUNK_W)],
            sem))
    for cp in copies:
        cp.wait()


@jax.jit
def kernel(time_seqs, seq_lengths, W, b):
    ts = time_seqs.astype(jnp.int32)
    sl = seq_lengths.astype(jnp.int32)
    wb = jnp.concatenate(
        [jnp.tile(W.reshape(-1), 2), jnp.tile(b.reshape(-1), 2)])
    out = _sc_embed(ts, sl, wb)
    return out.reshape(B, L, D)


# speedup vs baseline: 1.2763x; 1.0003x over previous
import functools

import jax
import jax.numpy as jnp
from jax import lax
from jax.experimental import pallas as pl
from jax.experimental.pallas import tpu as pltpu
from jax.experimental.pallas import tpu_sc as plsc

B = 16
L = 4096
D = 8
NC = 2
HALF = L // 2
OUT_W = HALF * D
N_CHUNKS = 4
CHUNK_BLKS = (HALF // 16) // N_CHUNKS
CHUNK_W = CHUNK_BLKS * 128

_mesh = plsc.VectorSubcoreMesh(core_axis_name="c", subcore_axis_name="s")


@functools.partial(
    pl.kernel,
    mesh=_mesh,
    out_type=jax.ShapeDtypeStruct((B, L * D // 128, 128), jnp.float32),
    scratch_types=[
        pltpu.VMEM((L,), jnp.int32),
        pltpu.VMEM((L,), jnp.float32),
        pltpu.VMEM((16,), jnp.int32),
        pltpu.VMEM((8,), jnp.float32),
        pltpu.VMEM((8,), jnp.float32),
        pltpu.VMEM((16,), jnp.float32),
        pltpu.VMEM((OUT_W // 128, 128), jnp.float32),
        pltpu.SemaphoreType.DMA,
    ],
    compiler_params=pltpu.CompilerParams(needs_layout_passes=False),
)
def _sc_embed(ts_hbm, len_hbm, w_hbm, b_hbm, out_hbm,
              ts_v, secs_v, len_v, w_v, b_v, red_v, out_v, sem):
    wid = lax.axis_index("s") * NC + lax.axis_index("c")
    row = wid // 2
    half = wid % 2

    c1 = pltpu.async_copy(ts_hbm.at[row], ts_v, sem)
    c2 = pltpu.async_copy(len_hbm, len_v, sem)
    c3 = pltpu.async_copy(w_hbm, w_v, sem)
    c4 = pltpu.async_copy(b_hbm, b_v, sem)
    c1.wait()
    c2.wait()
    c3.wait()
    c4.wait()

    iota = lax.iota(jnp.int32, 16)
    lane_pair = lax.shift_right_logical(iota, 3)
    lane_ch = lax.bitwise_and(iota, 7)
    inv_day = jnp.float32(1.0 / 86400.0)

    inf_v = jnp.full((16,), jnp.inf, jnp.float32)

    @plsc.parallel_loop(0, L // 16, unroll=8, carry=(inf_v, -inf_v))
    def _pass1(j, carry):
        mn_v, mx_v = carry
        ts = ts_v[pl.ds(j * 16, 16)]
        xf = ts.astype(jnp.float32)
        q = (xf * inv_day).astype(jnp.int32)
        qm = ((q.astype(jnp.float32) * 675.0) * 128.0).astype(jnp.int32)
        r = ts - qm
        r = jnp.where(r < 0, r + 86400, r)
        r = jnp.where(r >= 86400, r - 86400, r)
        secs = r.astype(jnp.float32)
        secs_v[pl.ds(j * 16, 16)] = secs
        return jnp.minimum(mn_v, secs), jnp.maximum(mx_v, secs)

    mn_v, mx_v = _pass1

    def lane_all_reduce(v, op):
        for step in (8, 4, 2, 1):
            red_v[...] = v
            v = op(v, plsc.load_gather(red_v, [lax.bitwise_xor(iota, step)]))
        return v

    mn = lane_all_reduce(mn_v, jnp.minimum)
    mx = lane_all_reduce(mx_v, jnp.maximum)

    wv = plsc.load_gather(w_v, [lane_ch])
    bv = plsc.load_gather(b_v, [lane_ch])
    scale = wv / (mx - mn)
    bias = bv - mn * scale
    row_len = plsc.load_gather(len_v, [jnp.broadcast_to(row, (16,))])

    base = half * HALF

    copies = []
    for c in range(N_CHUNKS):

        @plsc.parallel_loop(c * CHUNK_BLKS, (c + 1) * CHUNK_BLKS, unroll=2)
        def _pass2(j):
            lj = base + j * 16
            for t in range(8):
                l_idx = lane_pair + (lj + 2 * t)
                s_g = plsc.load_gather(secs_v, [l_idx])
                o = s_g * scale + bias
                o = jnp.where(l_idx < row_len, o, 0.0)
                out_v[j, pl.ds(t * 16, 16)] = o

        rows0 = c * CHUNK_BLKS
        copies.append(pltpu.async_copy(
            out_v.at[pl.ds(rows0, CHUNK_BLKS), :],
            out_hbm.at[row, pl.ds(half * (OUT_W // 128) + rows0, CHUNK_BLKS), :],
            sem))
    for cp in copies:
        cp.wait()


@jax.jit
def kernel(time_seqs, seq_lengths, W, b):
    ts = time_seqs.astype(jnp.int32)
    sl = seq_lengths.astype(jnp.int32)
    return _sc_embed(ts, sl, W.reshape(-1), b).reshape(B, L, D)


# speedup vs baseline: 2.5158x; 1.9712x over previous
import functools

import jax
import jax.numpy as jnp
from jax import lax
from jax.experimental import pallas as pl
from jax.experimental.pallas import tpu as pltpu
from jax.experimental.pallas import tpu_sc as plsc

B = 16
L = 4096
D = 8
NC = 2
NQ = 2
QW = L // NQ
_mesh = plsc.VectorSubcoreMesh(core_axis_name="c", subcore_axis_name="s")


@functools.partial(
    pl.kernel,
    mesh=_mesh,
    out_type=jax.ShapeDtypeStruct((B, D, L), jnp.float32),
    scratch_types=[
        pltpu.VMEM((L,), jnp.int32),
        pltpu.VMEM((L,), jnp.float32),
        pltpu.VMEM((16,), jnp.int32),
        pltpu.VMEM((8,), jnp.float32),
        pltpu.VMEM((8,), jnp.float32),
        pltpu.VMEM((16,), jnp.float32),
        pltpu.VMEM((4 * L,), jnp.float32),
        pltpu.SemaphoreType.DMA,
    ],
    compiler_params=pltpu.CompilerParams(
        needs_layout_passes=False, skip_device_barrier=True),
)
def _sc_embed(ts_hbm, len_hbm, w_hbm, b_hbm, out_hbm,
              ts_v, secs_v, len_v, w_v, b_v, red_v, out_v, sem):
    wid = lax.axis_index("s") * NC + lax.axis_index("c")
    row = wid // 2
    dg = wid % 2

    c1 = pltpu.async_copy(ts_hbm.at[row], ts_v, sem)
    c2 = pltpu.async_copy(len_hbm, len_v, sem)
    c3 = pltpu.async_copy(w_hbm, w_v, sem)
    c4 = pltpu.async_copy(b_hbm, b_v, sem)
    c1.wait()
    c2.wait()
    c3.wait()
    c4.wait()

    iota = lax.iota(jnp.int32, 16)
    inv_day = jnp.float32(1.0 / 86400.0)

    inf_v = jnp.full((16,), jnp.inf, jnp.float32)

    @plsc.parallel_loop(0, L // 16, unroll=2, carry=(inf_v, -inf_v))
    def _pass1(j, carry):
        mn_v, mx_v = carry
        ts = ts_v[pl.ds(j * 16, 16)]
        xf = ts.astype(jnp.float32)
        q = (xf * inv_day).astype(jnp.int32)
        qm = ((q.astype(jnp.float32) * 675.0) * 128.0).astype(jnp.int32)
        r = ts - qm
        r = jnp.where(r < 0, r + 86400, r)
        r = jnp.where(r >= 86400, r - 86400, r)
        secs = r.astype(jnp.float32)
        secs_v[pl.ds(j * 16, 16)] = secs
        return jnp.minimum(mn_v, secs), jnp.maximum(mx_v, secs)

    mn_v, mx_v = _pass1

    def lane_all_reduce(v, op):
        for step in (8, 4, 2, 1):
            red_v[...] = v
            v = op(v, plsc.load_gather(red_v, [lax.bitwise_xor(iota, step)]))
        return v

    mn = lane_all_reduce(mn_v, jnp.minimum)
    mx = lane_all_reduce(mx_v, jnp.maximum)
    inv_span = 1.0 / (mx - mn)
    row_len = plsc.load_gather(len_v, [jnp.broadcast_to(row, (16,))])

    scales = []
    biases = []
    for dloc in range(4):
        d = dg * 4 + dloc
        w_d = plsc.load_gather(w_v, [jnp.broadcast_to(d, (16,))])
        b_d = plsc.load_gather(b_v, [jnp.broadcast_to(d, (16,))])
        s_d = w_d * inv_span
        scales.append(s_d)
        biases.append(b_d - mn * s_d)

    copies = []
    for qq in range(NQ):

        @plsc.parallel_loop(qq * (QW // 16), (qq + 1) * (QW // 16), unroll=2)
        def _pass2(j):
            s = secs_v[pl.ds(j * 16, 16)]
            m = (iota + j * 16) < row_len
            for dloc in range(4):
                o = s * scales[dloc] + biases[dloc]
                o = jnp.where(m, o, 0.0)
                out_v[pl.ds(dloc * L + j * 16, 16)] = o

        for dloc in range(4):
            copies.append(pltpu.async_copy(
                out_v.at[pl.ds(dloc * L + qq * QW, QW)],
                out_hbm.at[row, dg * 4 + dloc, pl.ds(qq * QW, QW)],
                sem))
    for cp in copies:
        cp.wait()


@jax.jit
def kernel(time_seqs, seq_lengths, W, b):
    ts = time_seqs.astype(jnp.int32)
    sl = seq_lengths.astype(jnp.int32)
    out = _sc_embed(ts, sl, W.reshape(-1), b)
    return out.transpose(0, 2, 1)
